# Initial kernel scaffold; baseline (speedup 1.0000x reference)
import jax
import jax.numpy as jnp
from jax.experimental import pallas as pl


def kernel(boxes, scores):
    raise NotImplementedError("write your pallas kernel here")


# speedup vs baseline: 97.4852x; 97.4852x over previous
import jax
import jax.numpy as jnp
from jax import lax
from jax.experimental import pallas as pl
from jax.experimental.pallas import tpu as pltpu

_TOP_N = 5000
_BLK = 128
_NBLK = 40
_PAD_N = _BLK * _NBLK
_THR = 0.7


def _col(v, eye):
    return lax.dot_general(eye, v, (((1,), (1,)), ((), ())),
                           preferred_element_type=jnp.float32)


def _iou(x1a, y1a, x2a, y2a, aa, x1b, y1b, x2b, y2b, ab):
    ix1 = jnp.maximum(x1a, x1b)
    iy1 = jnp.maximum(y1a, y1b)
    ix2 = jnp.minimum(x2a, x2b)
    iy2 = jnp.minimum(y2a, y2b)
    iw = jnp.maximum(ix2 - ix1, 0.0)
    ih = jnp.maximum(iy2 - iy1, 0.0)
    inter = iw * ih
    union = aa + ab - inter
    return inter / (union + 1e-8)


def _nms_body(rawt_ref, sc_ref, out_ref, c_ref, keep_ref):
    rx = rawt_ref[0:1, :]
    ry = rawt_ref[1:2, :]
    rw = rawt_ref[2:3, :]
    rh = rawt_ref[3:4, :]
    cx = rx * 1000.0
    cy = ry * 1000.0
    w = rw * 200.0 + 1.0
    h = rh * 200.0 + 1.0
    c_ref[0:1, :] = cx - 0.5 * w
    c_ref[1:2, :] = cy - 0.5 * h
    c_ref[2:3, :] = cx + 0.5 * w
    c_ref[3:4, :] = cy + 0.5 * h
    c_ref[4:5, :] = (c_ref[2:3, :] - c_ref[0:1, :]) * (c_ref[3:4, :] - c_ref[1:2, :])
    keep_ref[...] = jnp.ones((1, _PAD_N), jnp.float32)

    r128 = lax.broadcasted_iota(jnp.int32, (_BLK, _BLK), 0)
    c128 = lax.broadcasted_iota(jnp.int32, (_BLK, _BLK), 1)
    eye = jnp.where(r128 == c128, 1.0, 0.0).astype(jnp.float32)
    upper = r128 < c128
    lane = lax.broadcasted_iota(jnp.int32, (1, _PAD_N), 1)

    x1f = c_ref[0:1, :]
    y1f = c_ref[1:2, :]
    x2f = c_ref[2:3, :]
    y2f = c_ref[3:4, :]
    arf = c_ref[4:5, :]

    for i in range(_NBLK):
        base = i * _BLK
        x1r = c_ref[0:1, base:base + _BLK]
        y1r = c_ref[1:2, base:base + _BLK]
        x2r = c_ref[2:3, base:base + _BLK]
        y2r = c_ref[3:4, base:base + _BLK]
        arr = c_ref[4:5, base:base + _BLK]
        x1c = _col(x1r, eye)
        y1c = _col(y1r, eye)
        x2c = _col(x2r, eye)
        y2c = _col(y2r, eye)
        arc = _col(arr, eye)

        iou_ii = _iou(x1c, y1c, x2c, y2c, arc, x1r, y1r, x2r, y2r, arr)
        E = jnp.where((iou_ii > _THR) & upper, 1.0, 0.0).astype(jnp.float32)
        inc = keep_ref[0:1, base:base + _BLK]

        def _cond(carry):
            return carry[1]

        def _body(carry):
            k = carry[0]
            cnt = lax.dot_general(k, E, (((1,), (0,)), ((), ())),
                                  preferred_element_type=jnp.float32)
            knew = jnp.where(cnt > 0.5, 0.0, inc)
            changed = jnp.sum(jnp.abs(knew - k)) > 0.0
            return (knew, changed)

        ki, _ = lax.while_loop(_cond, _body, (inc, jnp.asarray(True)))
        keep_ref[0:1, base:base + _BLK] = ki

        if i + 1 < _NBLK:
            iou_cross = _iou(x1c, y1c, x2c, y2c, arc, x1f, y1f, x2f, y2f, arf)
            M = jnp.where(iou_cross > _THR, 1.0, 0.0).astype(jnp.float32)
            cnt = lax.dot_general(ki, M, (((1,), (0,)), ((), ())),
                                  preferred_element_type=jnp.float32)
            sup = (cnt > 0.5) & (lane >= base + _BLK)
            keep_ref[...] = jnp.where(sup, 0.0, keep_ref[...])

    k = keep_ref[...]
    out_ref[0:1, :] = c_ref[0:1, :] * k
    out_ref[1:2, :] = c_ref[1:2, :] * k
    out_ref[2:3, :] = c_ref[2:3, :] * k
    out_ref[3:4, :] = c_ref[3:4, :] * k
    out_ref[4:5, :] = sc_ref[...] * k
    out_ref[5:8, :] = jnp.zeros((3, _PAD_N), jnp.float32)


_nms_call = pl.pallas_call(
    _nms_body,
    out_shape=jax.ShapeDtypeStruct((8, _PAD_N), jnp.float32),
    scratch_shapes=[
        pltpu.VMEM((8, _PAD_N), jnp.float32),
        pltpu.VMEM((1, _PAD_N), jnp.float32),
    ],
)


def kernel(boxes, scores):
    top_scores, idx = lax.top_k(scores, _TOP_N)
    raw = jnp.take(boxes, idx, axis=0)
    rawt = jnp.pad(raw, ((0, _PAD_N - _TOP_N), (0, 0))).T
    sct = jnp.pad(top_scores, (0, _PAD_N - _TOP_N))[None, :]
    out8 = _nms_call(rawt, sct)
    return out8[:5, :_TOP_N].T
